# baseline (speedup 1.0000x reference)
import jax
import jax.numpy as jnp
from jax import lax
from jax.experimental import pallas as pl
from jax.experimental.pallas import tpu as pltpu
from jax.experimental.pallas import tpu_sc as plsc

B = 16384
D = 64
NW = 32
PPW = B // NW
CT_TOTAL = 7813
CT_PER_W = 245
BOXCAP = 128
NSLOT = 2 * B
_PARAMS = pltpu.CompilerParams(
    needs_layout_passes=False, use_tc_tiling_on_sc=True)


def _wid():
    return lax.axis_index("s") * 2 + lax.axis_index("c")


def _extract(vec, j):
    return jnp.sum(jnp.where(lax.iota(jnp.int32, 16) == j, vec, 0))


def _bin_body(src_hbm, tgt_hbm, boxes_hbm, counts_hbm, idxv, outbox, cntv,
              sem):
    w = _wid()
    pltpu.sync_copy(src_hbm.at[pl.ds(w * PPW, PPW)], idxv.at[pl.ds(0, PPW)])
    pltpu.sync_copy(tgt_hbm.at[pl.ds(w * PPW, PPW)],
                    idxv.at[pl.ds(PPW, PPW)])

    lanes = lax.iota(jnp.int32, 16)
    for k in range(NW):
        def scan_body(i, cnt, k=k):
            v = idxv[pl.ds(i * 16, 16)]
            ct = lax.shift_right_logical(v, 7)
            m = jnp.bitwise_and(ct, NW - 1) == k
            local = i * 16 + lanes
            slot = jnp.where(local < PPW, w * PPW + local,
                             B + w * PPW + (local - PPW))
            comb = (lax.shift_right_logical(ct, 5) << 22) | \
                   (jnp.bitwise_and(v, 127) << 15) | slot
            plsc.store_compressed(
                outbox.at[pl.ds(k * BOXCAP + cnt, 16)], comb, mask=m)
            return cnt + plsc.all_reduce_population_count(m)[0]

        cnt_k = lax.fori_loop(0, (2 * PPW) // 16, scan_body, 0)
        half = k // 16
        cntv[pl.ds(half * 16, 16)] = jnp.where(
            lanes == (k % 16), cnt_k, cntv[pl.ds(half * 16, 16)])

    pltpu.sync_copy(outbox, boxes_hbm.at[pl.ds(w * NW * BOXCAP, NW * BOXCAP)])
    pltpu.sync_copy(cntv, counts_hbm.at[pl.ds(w * NW, NW)])


@jax.jit
def _bin_call(sources, targets):
    f = pl.kernel(
        _bin_body,
        mesh=plsc.VectorSubcoreMesh(core_axis_name="c", subcore_axis_name="s"),
        out_type=(jax.ShapeDtypeStruct((NW * NW * BOXCAP,), jnp.int32),
                  jax.ShapeDtypeStruct((NW * NW,), jnp.int32)),
        scratch_types=[
            pltpu.VMEM((2 * PPW,), jnp.int32),
            pltpu.VMEM((NW * BOXCAP,), jnp.int32),
            pltpu.VMEM((NW,), jnp.int32),
            pltpu.SemaphoreType.DMA,
        ],
        compiler_params=_PARAMS,
    )
    return f(sources, targets)


WLCAP = 2048
NBKT = 16
BKTCAP = 128


def _sweep_body(table_hbm, boxes_hbm, counts_hbm, stage_hbm,
                wl, cntv, bkt, bktcnt, piece, col, sem, csem):
    k = _wid()
    lanes = lax.iota(jnp.int32, 16)

    pltpu.sync_copy(counts_hbm, cntv)
    for w in range(NW):
        pltpu.sync_copy(
            boxes_hbm.at[pl.ds((w * NW + k) * BOXCAP, BOXCAP)],
            wl.at[pl.ds(w * BOXCAP, BOXCAP)])

    def bucket_pass(b):
        def inner(w, cnt, b=b):
            mycnt = _extract(cntv[pl.ds((w * NW + k) // 16 * 16, 16)],
                             (w * NW + k) % 16)
            def vreg_body(q, cnt2):
                v = wl[pl.ds(w * BOXCAP + q * 16, 16)]
                valid = (q * 16 + lanes) < mycnt
                m = jnp.logical_and(
                    valid,
                    lax.shift_right_logical(v, 22 + 4) == b)
                plsc.store_compressed(
                    bkt.at[pl.ds(b * BKTCAP + cnt2, 16)], v, mask=m)
                return cnt2 + plsc.all_reduce_population_count(m)[0]
            return lax.fori_loop(0, BOXCAP // 16, vreg_body, cnt)
        return lax.fori_loop(0, NW, inner, 0)

    for b in range(NBKT):
        nb = bucket_pass(b)
        half = b // 16
        bktcnt[pl.ds(0, 16)] = jnp.where(lanes == b, nb, bktcnt[pl.ds(0, 16)])

    def fire(ct_i, which):
        ct = ct_i * NW + k
        @pl.when(ct < CT_TOTAL - 1)
        def _():
            for bb in range(8):
                pltpu.async_copy(
                    table_hbm.at[pl.ds(bb * 8, 8), pl.ds(ct * 128, 128)],
                    piece.at[which, pl.ds(bb * 8, 8), :], sem.at[which])
        @pl.when(ct == CT_TOTAL - 1)
        def _():
            for bb in range(8):
                pltpu.async_copy(
                    table_hbm.at[pl.ds(bb * 8, 8), pl.ds(ct * 128, 64)],
                    piece.at[which, pl.ds(bb * 8, 8), pl.ds(0, 64)],
                    sem.at[which])

    fire(0, 0)

    def ct_body(ct_i, hitcount):
        which = lax.rem(ct_i, 2)
        ct = ct_i * NW + k

        @pl.when(ct_i + 1 < CT_PER_W)
        def _():
            fire(ct_i + 1, lax.rem(ct_i + 1, 2))

        @pl.when(ct < CT_TOTAL - 1)
        def _():
            for bb in range(8):
                pltpu.make_async_copy(
                    table_hbm.at[pl.ds(0, 8), pl.ds(0, 128)],
                    piece.at[0].at[pl.ds(bb * 8, 8), :],
                    sem.at[which]).wait()

        @pl.when(ct == CT_TOTAL - 1)
        def _():
            for bb in range(8):
                pltpu.make_async_copy(
                    table_hbm.at[pl.ds(0, 8), pl.ds(0, 64)],
                    piece.at[0].at[pl.ds(bb * 8, 8), pl.ds(0, 64)],
                    sem.at[which]).wait()

        def handle_hits(hc):
            b = lax.shift_right_logical(ct_i, 4)
            nb = _extract(bktcnt[pl.ds(0, 16)], b)

            def vreg_hits(q, hc2):
                v = bkt[pl.ds(b * BKTCAP + q * 16, 16)]
                valid = (q * 16 + lanes) < nb
                m = jnp.logical_and(valid,
                                    lax.shift_right_logical(v, 22) ==
                                    lax.shift_right_logical(ct, 5))

                def while_cond(carry):
                    mm, _ = carry
                    return jnp.any(mm)

                def while_body(carry):
                    mm, h = carry
                    j = plsc.all_reduce_ffs(mm)[0]
                    comb = _extract(v, j)
                    cc = jnp.bitwise_and(
                        lax.shift_right_logical(comb, 15), 127)
                    slot = jnp.bitwise_and(comb, NSLOT - 1)
                    ring = lax.rem(h, 16)
                    ccv = jnp.full((16,), cc, jnp.int32)
                    for q4 in range(4):
                        g = plsc.load_gather(
                            piece.at[which],
                            [lanes + q4 * 16, ccv])
                        col[ring, pl.ds(q4 * 16, 16)] = g
                    pltpu.async_copy(
                        col.at[ring],
                        stage_hbm.at[pl.ds(slot * D, D)], csem)
                    @pl.when(h >= 16)
                    def _():
                        pltpu.make_async_copy(
                            stage_hbm.at[pl.ds(0, D)], col.at[0],
                            csem).wait()
                    return (jnp.logical_and(mm, lanes != j), h + 1)

                mm, h2 = lax.while_loop(while_cond, while_body, (m, hc2))
                return h2

            return lax.fori_loop(0, BKTCAP // 16, vreg_hits, hc)

        return handle_hits(hitcount)

    total_hits = lax.fori_loop(0, CT_PER_W, ct_body, 0)

    def drain(i, c):
        @pl.when(i < jnp.minimum(total_hits, 16))
        def _():
            pltpu.make_async_copy(
                stage_hbm.at[pl.ds(0, D)], col.at[0], csem).wait()
        return c

    lax.fori_loop(0, 16, drain, 0)


@jax.jit
def _sweep_call(table_t, boxes, counts):
    f = pl.kernel(
        _sweep_body,
        mesh=plsc.VectorSubcoreMesh(core_axis_name="c", subcore_axis_name="s"),
        out_type=jax.ShapeDtypeStruct((NSLOT * D,), jnp.float32),
        scratch_types=[
            pltpu.VMEM((NW * BOXCAP,), jnp.int32),
            pltpu.VMEM((NW * NW,), jnp.int32),
            pltpu.VMEM((NBKT * BKTCAP,), jnp.int32),
            pltpu.VMEM((16,), jnp.int32),
            pltpu.VMEM((2, D, 128), jnp.float32),
            pltpu.VMEM((16, D), jnp.float32),
            pltpu.SemaphoreType.DMA((2,)),
            pltpu.SemaphoreType.DMA,
        ],
        compiler_params=_PARAMS,
    )
    return f(table_t, boxes, counts)


def _dot_body(stage_hbm, wb_hbm, out_hbm, srows, trows, outv, wbv, sem):
    w = _wid()
    base = w * PPW
    pltpu.sync_copy(stage_hbm.at[pl.ds(base * D, PPW * D)], srows)
    pltpu.sync_copy(stage_hbm.at[pl.ds((B + base) * D, PPW * D)], trows)
    pltpu.sync_copy(wb_hbm, wbv)
    wv = wbv[...]
    wgt = wv[0]
    bb = wv[1]
    lanes = lax.iota(jnp.int32, 16)

    def group_body(g, carry):
        acc = jnp.zeros((16,), jnp.float32)
        for r in range(16):
            i = (g * 16 + r) * D
            a0 = srows[pl.ds(i, 16)] * trows[pl.ds(i, 16)]
            a1 = srows[pl.ds(i + 16, 16)] * trows[pl.ds(i + 16, 16)]
            a2 = srows[pl.ds(i + 32, 16)] * trows[pl.ds(i + 32, 16)]
            a3 = srows[pl.ds(i + 48, 16)] * trows[pl.ds(i + 48, 16)]
            s = jnp.sum((a0 + a1) + (a2 + a3))
            acc = jnp.where(lanes == r, s, acc)
        outv[pl.ds(g * 16, 16)] = acc * wgt + bb
        return carry

    lax.fori_loop(0, PPW // 16, group_body, 0)
    pltpu.sync_copy(outv, out_hbm.at[pl.ds(base, PPW)])


@jax.jit
def _dot_call(stage, wb):
    f = pl.kernel(
        _dot_body,
        mesh=plsc.VectorSubcoreMesh(core_axis_name="c", subcore_axis_name="s"),
        out_type=jax.ShapeDtypeStruct((B,), jnp.float32),
        scratch_types=[
            pltpu.VMEM((PPW * D,), jnp.float32),
            pltpu.VMEM((PPW * D,), jnp.float32),
            pltpu.VMEM((PPW,), jnp.float32),
            pltpu.VMEM((16,), jnp.float32),
            pltpu.SemaphoreType.DMA,
        ],
        compiler_params=_PARAMS,
    )
    return f(stage, wb)


def kernel(sources, targets, table, W, b):
    wb = jnp.zeros((16,), jnp.float32)
    wb = wb.at[0].set(W.reshape(())).at[1].set(b.reshape(()))
    boxes, counts = _bin_call(sources, targets)
    stage = _sweep_call(table.T, boxes, counts)
    out = _dot_call(stage, wb)
    return out.reshape(B, 1)


# speedup vs baseline: 1.0349x; 1.0349x over previous
import jax
import jax.numpy as jnp
from jax import lax
from jax.experimental import pallas as pl
from jax.experimental.pallas import tpu as pltpu
from jax.experimental.pallas import tpu_sc as plsc

B = 16384
D = 64
NW = 32
PPW = B // NW
CT_TOTAL = 7813
CT_PER_W = 245
BOXCAP = 128
NSLOT = 2 * B
_PARAMS = pltpu.CompilerParams(
    needs_layout_passes=False, use_tc_tiling_on_sc=True)


def _wid():
    return lax.axis_index("s") * 2 + lax.axis_index("c")


def _extract(vec, j):
    return vec[jnp.full((16,), j, jnp.int32)][0]


def _bin_body(src_hbm, tgt_hbm, boxes_hbm, counts_hbm, idxv, outbox, cntv,
              sem):
    w = _wid()
    pltpu.sync_copy(src_hbm.at[pl.ds(w * PPW, PPW)], idxv.at[pl.ds(0, PPW)])
    pltpu.sync_copy(tgt_hbm.at[pl.ds(w * PPW, PPW)],
                    idxv.at[pl.ds(PPW, PPW)])

    lanes = lax.iota(jnp.int32, 16)
    for k in range(NW):
        def scan_body(i, cnt, k=k):
            v = idxv[pl.ds(i * 16, 16)]
            ct = lax.shift_right_logical(v, 7)
            m = jnp.bitwise_and(ct, NW - 1) == k
            local = i * 16 + lanes
            slot = jnp.where(local < PPW, w * PPW + local,
                             B + w * PPW + (local - PPW))
            comb = (lax.shift_right_logical(ct, 5) << 22) | \
                   (jnp.bitwise_and(v, 127) << 15) | slot
            plsc.store_compressed(
                outbox.at[pl.ds(k * BOXCAP + cnt, 16)], comb, mask=m)
            return cnt + plsc.all_reduce_population_count(m)[0]

        cnt_k = lax.fori_loop(0, (2 * PPW) // 16, scan_body, 0)
        half = k // 16
        cntv[pl.ds(half * 16, 16)] = jnp.where(
            lanes == (k % 16), cnt_k, cntv[pl.ds(half * 16, 16)])

    pltpu.sync_copy(outbox, boxes_hbm.at[pl.ds(w * NW * BOXCAP, NW * BOXCAP)])
    pltpu.sync_copy(cntv, counts_hbm.at[pl.ds(w * NW, NW)])


@jax.jit
def _bin_call(sources, targets):
    f = pl.kernel(
        _bin_body,
        mesh=plsc.VectorSubcoreMesh(core_axis_name="c", subcore_axis_name="s"),
        out_type=(jax.ShapeDtypeStruct((NW * NW * BOXCAP,), jnp.int32),
                  jax.ShapeDtypeStruct((NW * NW,), jnp.int32)),
        scratch_types=[
            pltpu.VMEM((2 * PPW,), jnp.int32),
            pltpu.VMEM((NW * BOXCAP,), jnp.int32),
            pltpu.VMEM((NW,), jnp.int32),
            pltpu.SemaphoreType.DMA,
        ],
        compiler_params=_PARAMS,
    )
    return f(sources, targets)


WLCAP = 2048
NBKT = 16
BKTCAP = 128
FBCAP = 48


def _sweep_body(table_hbm, boxes_hbm, counts_hbm, stage_hbm,
                wl, cntv, bkt, bktcnt, fbkt, fbcnt, piece, col, sem, csem):
    k = _wid()
    lanes = lax.iota(jnp.int32, 16)

    pltpu.sync_copy(counts_hbm, cntv)
    for w in range(NW):
        pltpu.sync_copy(
            boxes_hbm.at[pl.ds((w * NW + k) * BOXCAP, BOXCAP)],
            wl.at[pl.ds(w * BOXCAP, BOXCAP)])

    def bucket_pass(b):
        def inner(w, cnt, b=b):
            mycnt = _extract(cntv[pl.ds((w * NW + k) // 16 * 16, 16)],
                             (w * NW + k) % 16)
            def vreg_body(q, cnt2):
                v = wl[pl.ds(w * BOXCAP + q * 16, 16)]
                valid = (q * 16 + lanes) < mycnt
                m = jnp.logical_and(
                    valid,
                    lax.shift_right_logical(v, 22 + 4) == b)
                plsc.store_compressed(
                    bkt.at[pl.ds(b * BKTCAP + cnt2, 16)], v, mask=m)
                return cnt2 + plsc.all_reduce_population_count(m)[0]
            return lax.fori_loop(0, BOXCAP // 16, vreg_body, cnt)
        return lax.fori_loop(0, NW, inner, 0)

    for b in range(NBKT):
        nb = bucket_pass(b)
        half = b // 16
        bktcnt[pl.ds(0, 16)] = jnp.where(lanes == b, nb, bktcnt[pl.ds(0, 16)])

    for b in range(NBKT):
        nbv = _extract(bktcnt[pl.ds(0, 16)], b)
        for sub in range(4):
            def sub_pass(q, cnt2, b=b, sub=sub, nbv=nbv):
                v = bkt[pl.ds(b * BKTCAP + q * 16, 16)]
                valid = (q * 16 + lanes) < nbv
                m = jnp.logical_and(
                    valid,
                    jnp.bitwise_and(lax.shift_right_logical(v, 24), 3) == sub)
                plsc.store_compressed(
                    fbkt.at[pl.ds((b * 4 + sub) * FBCAP + cnt2, 16)],
                    v, mask=m)
                return cnt2 + plsc.all_reduce_population_count(m)[0]
            nf = lax.fori_loop(0, BKTCAP // 16, sub_pass, 0)
            fb = b * 4 + sub
            half2 = fb // 16
            fbcnt[pl.ds(half2 * 16, 16)] = jnp.where(
                lanes == (fb % 16), nf, fbcnt[pl.ds(half2 * 16, 16)])

    def fire(ct_i, which):
        ct = ct_i * NW + k
        @pl.when(ct < CT_TOTAL - 1)
        def _():
            for bb in range(8):
                pltpu.async_copy(
                    table_hbm.at[pl.ds(bb * 8, 8), pl.ds(ct * 128, 128)],
                    piece.at[which, pl.ds(bb * 8, 8), :], sem.at[which])
        @pl.when(ct == CT_TOTAL - 1)
        def _():
            for bb in range(8):
                pltpu.async_copy(
                    table_hbm.at[pl.ds(bb * 8, 8), pl.ds(ct * 128, 64)],
                    piece.at[which, pl.ds(bb * 8, 8), pl.ds(0, 64)],
                    sem.at[which])

    fire(0, 0)

    def ct_body(ct_i, hitcount):
        which = lax.rem(ct_i, 2)
        ct = ct_i * NW + k

        @pl.when(ct_i + 1 < CT_PER_W)
        def _():
            fire(ct_i + 1, lax.rem(ct_i + 1, 2))

        @pl.when(ct < CT_TOTAL - 1)
        def _():
            for bb in range(8):
                pltpu.make_async_copy(
                    table_hbm.at[pl.ds(0, 8), pl.ds(0, 128)],
                    piece.at[0].at[pl.ds(bb * 8, 8), :],
                    sem.at[which]).wait()

        @pl.when(ct == CT_TOTAL - 1)
        def _():
            for bb in range(8):
                pltpu.make_async_copy(
                    table_hbm.at[pl.ds(0, 8), pl.ds(0, 64)],
                    piece.at[0].at[pl.ds(bb * 8, 8), pl.ds(0, 64)],
                    sem.at[which]).wait()

        def handle_hits(hc):
            b = lax.shift_right_logical(ct_i, 2)
            nb = _extract(
                fbcnt[pl.ds(lax.shift_right_logical(b, 4) * 16, 16)],
                jnp.bitwise_and(b, 15))

            def vreg_hits(q, hc2):
                v = fbkt[pl.ds(b * FBCAP + q * 16, 16)]
                valid = (q * 16 + lanes) < nb
                m = jnp.logical_and(valid,
                                    lax.shift_right_logical(v, 22) ==
                                    lax.shift_right_logical(ct, 5))

                def while_cond(carry):
                    mm, _ = carry
                    return jnp.any(mm)

                def while_body(carry):
                    mm, h = carry
                    j = plsc.all_reduce_ffs(mm)[0]
                    comb = _extract(v, j)
                    cc = jnp.bitwise_and(
                        lax.shift_right_logical(comb, 15), 127)
                    slot = jnp.bitwise_and(comb, NSLOT - 1)
                    ring = lax.rem(h, 16)
                    ccv = jnp.full((16,), cc, jnp.int32)
                    for q4 in range(4):
                        g = plsc.load_gather(
                            piece.at[which],
                            [lanes + q4 * 16, ccv])
                        col[ring, pl.ds(q4 * 16, 16)] = g
                    pltpu.async_copy(
                        col.at[ring],
                        stage_hbm.at[pl.ds(slot * D, D)], csem)
                    @pl.when(h >= 16)
                    def _():
                        pltpu.make_async_copy(
                            stage_hbm.at[pl.ds(0, D)], col.at[0],
                            csem).wait()
                    return (jnp.logical_and(mm, lanes != j), h + 1)

                mm, h2 = lax.while_loop(while_cond, while_body, (m, hc2))
                return h2

            return lax.fori_loop(0, FBCAP // 16, vreg_hits, hc)

        return handle_hits(hitcount)

    total_hits = lax.fori_loop(0, CT_PER_W, ct_body, 0)

    def drain(i, c):
        @pl.when(i < jnp.minimum(total_hits, 16))
        def _():
            pltpu.make_async_copy(
                stage_hbm.at[pl.ds(0, D)], col.at[0], csem).wait()
        return c

    lax.fori_loop(0, 16, drain, 0)


@jax.jit
def _sweep_call(table_t, boxes, counts):
    f = pl.kernel(
        _sweep_body,
        mesh=plsc.VectorSubcoreMesh(core_axis_name="c", subcore_axis_name="s"),
        out_type=jax.ShapeDtypeStruct((NSLOT * D,), jnp.float32),
        scratch_types=[
            pltpu.VMEM((NW * BOXCAP,), jnp.int32),
            pltpu.VMEM((NW * NW,), jnp.int32),
            pltpu.VMEM((NBKT * BKTCAP,), jnp.int32),
            pltpu.VMEM((16,), jnp.int32),
            pltpu.VMEM((64 * FBCAP,), jnp.int32),
            pltpu.VMEM((64,), jnp.int32),
            pltpu.VMEM((2, D, 128), jnp.float32),
            pltpu.VMEM((16, D), jnp.float32),
            pltpu.SemaphoreType.DMA((2,)),
            pltpu.SemaphoreType.DMA,
        ],
        compiler_params=_PARAMS,
    )
    return f(table_t, boxes, counts)


def _dot_body(stage_hbm, wb_hbm, out_hbm, srows, trows, outv, wbv, sem):
    w = _wid()
    base = w * PPW
    pltpu.sync_copy(stage_hbm.at[pl.ds(base * D, PPW * D)], srows)
    pltpu.sync_copy(stage_hbm.at[pl.ds((B + base) * D, PPW * D)], trows)
    pltpu.sync_copy(wb_hbm, wbv)
    wv = wbv[...]
    wgt = wv[0]
    bb = wv[1]
    lanes = lax.iota(jnp.int32, 16)

    def group_body(g, carry):
        acc = jnp.zeros((16,), jnp.float32)
        for r in range(16):
            i = (g * 16 + r) * D
            a0 = srows[pl.ds(i, 16)] * trows[pl.ds(i, 16)]
            a1 = srows[pl.ds(i + 16, 16)] * trows[pl.ds(i + 16, 16)]
            a2 = srows[pl.ds(i + 32, 16)] * trows[pl.ds(i + 32, 16)]
            a3 = srows[pl.ds(i + 48, 16)] * trows[pl.ds(i + 48, 16)]
            s = jnp.sum((a0 + a1) + (a2 + a3))
            acc = jnp.where(lanes == r, s, acc)
        outv[pl.ds(g * 16, 16)] = acc * wgt + bb
        return carry

    lax.fori_loop(0, PPW // 16, group_body, 0)
    pltpu.sync_copy(outv, out_hbm.at[pl.ds(base, PPW)])


@jax.jit
def _dot_call(stage, wb):
    f = pl.kernel(
        _dot_body,
        mesh=plsc.VectorSubcoreMesh(core_axis_name="c", subcore_axis_name="s"),
        out_type=jax.ShapeDtypeStruct((B,), jnp.float32),
        scratch_types=[
            pltpu.VMEM((PPW * D,), jnp.float32),
            pltpu.VMEM((PPW * D,), jnp.float32),
            pltpu.VMEM((PPW,), jnp.float32),
            pltpu.VMEM((16,), jnp.float32),
            pltpu.SemaphoreType.DMA,
        ],
        compiler_params=_PARAMS,
    )
    return f(stage, wb)


def kernel(sources, targets, table, W, b):
    wb = jnp.zeros((16,), jnp.float32)
    wb = wb.at[0].set(W.reshape(())).at[1].set(b.reshape(()))
    boxes, counts = _bin_call(sources, targets)
    stage = _sweep_call(table.T, boxes, counts)
    out = _dot_call(stage, wb)
    return out.reshape(B, 1)


# speedup vs baseline: 1.2537x; 1.2113x over previous
import jax
import jax.numpy as jnp
from jax import lax
from jax.experimental import pallas as pl
from jax.experimental.pallas import tpu as pltpu
from jax.experimental.pallas import tpu_sc as plsc

B = 16384
D = 64
NW = 32
PPW = B // NW
CT_TOTAL = 7813
CT_PER_W = 245
BOXCAP = 128
NSLOT = 2 * B
_PARAMS = pltpu.CompilerParams(
    needs_layout_passes=False, use_tc_tiling_on_sc=True)


def _wid():
    return lax.axis_index("s") * 2 + lax.axis_index("c")


def _extract(vec, j):
    return vec[jnp.full((16,), j, jnp.int32)][0]


def _bin_body(src_hbm, tgt_hbm, boxes_hbm, counts_hbm, idxv, outbox, cntv,
              sem):
    w = _wid()
    pltpu.sync_copy(src_hbm.at[pl.ds(w * PPW, PPW)], idxv.at[pl.ds(0, PPW)])
    pltpu.sync_copy(tgt_hbm.at[pl.ds(w * PPW, PPW)],
                    idxv.at[pl.ds(PPW, PPW)])

    lanes = lax.iota(jnp.int32, 16)
    for k in range(NW):
        def scan_body(i, cnt, k=k):
            v = idxv[pl.ds(i * 16, 16)]
            ct = lax.shift_right_logical(v, 7)
            m = jnp.bitwise_and(ct, NW - 1) == k
            local = i * 16 + lanes
            slot = jnp.where(local < PPW, w * PPW + local,
                             B + w * PPW + (local - PPW))
            comb = (lax.shift_right_logical(ct, 5) << 22) | \
                   (jnp.bitwise_and(v, 127) << 15) | slot
            plsc.store_compressed(
                outbox.at[pl.ds(k * BOXCAP + cnt, 16)], comb, mask=m)
            return cnt + plsc.all_reduce_population_count(m)[0]

        cnt_k = lax.fori_loop(0, (2 * PPW) // 16, scan_body, 0)
        half = k // 16
        cntv[pl.ds(half * 16, 16)] = jnp.where(
            lanes == (k % 16), cnt_k, cntv[pl.ds(half * 16, 16)])

    pltpu.sync_copy(outbox, boxes_hbm.at[pl.ds(w * NW * BOXCAP, NW * BOXCAP)])
    pltpu.sync_copy(cntv, counts_hbm.at[pl.ds(w * NW, NW)])


@jax.jit
def _bin_call(sources, targets):
    f = pl.kernel(
        _bin_body,
        mesh=plsc.VectorSubcoreMesh(core_axis_name="c", subcore_axis_name="s"),
        out_type=(jax.ShapeDtypeStruct((NW * NW * BOXCAP,), jnp.int32),
                  jax.ShapeDtypeStruct((NW * NW,), jnp.int32)),
        scratch_types=[
            pltpu.VMEM((2 * PPW,), jnp.int32),
            pltpu.VMEM((NW * BOXCAP,), jnp.int32),
            pltpu.VMEM((NW,), jnp.int32),
            pltpu.SemaphoreType.DMA,
        ],
        compiler_params=_PARAMS,
    )
    return f(sources, targets)


WLCAP = 2048
NBKT = 16
BKTCAP = 128
FBCAP = 48


def _sweep_body(table_hbm, boxes_hbm, counts_hbm, stage_hbm,
                wl, cntv, bkt, bktcnt, fbkt, fbcnt, piece, col, sem, csem):
    k = _wid()
    lanes = lax.iota(jnp.int32, 16)

    pltpu.sync_copy(counts_hbm, cntv)
    for w in range(NW):
        pltpu.sync_copy(
            boxes_hbm.at[pl.ds((w * NW + k) * BOXCAP, BOXCAP)],
            wl.at[pl.ds(w * BOXCAP, BOXCAP)])

    def bucket_pass(b):
        def inner(w, cnt, b=b):
            mycnt = _extract(cntv[pl.ds((w * NW + k) // 16 * 16, 16)],
                             (w * NW + k) % 16)
            def vreg_body(q, cnt2):
                v = wl[pl.ds(w * BOXCAP + q * 16, 16)]
                valid = (q * 16 + lanes) < mycnt
                m = jnp.logical_and(
                    valid,
                    lax.shift_right_logical(v, 22 + 4) == b)
                plsc.store_compressed(
                    bkt.at[pl.ds(b * BKTCAP + cnt2, 16)], v, mask=m)
                return cnt2 + plsc.all_reduce_population_count(m)[0]
            return lax.fori_loop(0, BOXCAP // 16, vreg_body, cnt)
        return lax.fori_loop(0, NW, inner, 0)

    for b in range(NBKT):
        nb = bucket_pass(b)
        half = b // 16
        bktcnt[pl.ds(0, 16)] = jnp.where(lanes == b, nb, bktcnt[pl.ds(0, 16)])

    for b in range(NBKT):
        nbv = _extract(bktcnt[pl.ds(0, 16)], b)
        for sub in range(4):
            def sub_pass(q, cnt2, b=b, sub=sub, nbv=nbv):
                v = bkt[pl.ds(b * BKTCAP + q * 16, 16)]
                valid = (q * 16 + lanes) < nbv
                m = jnp.logical_and(
                    valid,
                    jnp.bitwise_and(lax.shift_right_logical(v, 24), 3) == sub)
                plsc.store_compressed(
                    fbkt.at[pl.ds((b * 4 + sub) * FBCAP + cnt2, 16)],
                    v, mask=m)
                return cnt2 + plsc.all_reduce_population_count(m)[0]
            nf = lax.fori_loop(0, BKTCAP // 16, sub_pass, 0)
            fb = b * 4 + sub
            half2 = fb // 16
            fbcnt[pl.ds(half2 * 16, 16)] = jnp.where(
                lanes == (fb % 16), nf, fbcnt[pl.ds(half2 * 16, 16)])

    def fire(ct_i, which):
        ct = ct_i * NW + k
        @pl.when(ct < CT_TOTAL - 1)
        def _():
            for bb in range(8):
                pltpu.async_copy(
                    table_hbm.at[pl.ds(bb * 8, 8), pl.ds(ct * 128, 128)],
                    piece.at[which, pl.ds(bb * 8, 8), :], sem.at[which])
        @pl.when(ct == CT_TOTAL - 1)
        def _():
            for bb in range(8):
                pltpu.async_copy(
                    table_hbm.at[pl.ds(bb * 8, 8), pl.ds(ct * 128, 64)],
                    piece.at[which, pl.ds(bb * 8, 8), pl.ds(0, 64)],
                    sem.at[which])

    fire(0, 0)
    fire(1, 1)
    fire(2, 2)

    def ct_body(ct_i, hitcount):
        which = lax.rem(ct_i, 4)
        ct = ct_i * NW + k

        @pl.when(ct_i + 3 < CT_PER_W)
        def _():
            fire(ct_i + 3, lax.rem(ct_i + 3, 4))

        @pl.when(ct < CT_TOTAL - 1)
        def _():
            for bb in range(8):
                pltpu.make_async_copy(
                    table_hbm.at[pl.ds(0, 8), pl.ds(0, 128)],
                    piece.at[0].at[pl.ds(bb * 8, 8), :],
                    sem.at[which]).wait()

        @pl.when(ct == CT_TOTAL - 1)
        def _():
            for bb in range(8):
                pltpu.make_async_copy(
                    table_hbm.at[pl.ds(0, 8), pl.ds(0, 64)],
                    piece.at[0].at[pl.ds(bb * 8, 8), pl.ds(0, 64)],
                    sem.at[which]).wait()

        def handle_hits(hc):
            b = lax.shift_right_logical(ct_i, 2)
            nb = _extract(
                fbcnt[pl.ds(lax.shift_right_logical(b, 4) * 16, 16)],
                jnp.bitwise_and(b, 15))

            def vreg_hits(q, hc2):
                v = fbkt[pl.ds(b * FBCAP + q * 16, 16)]
                valid = (q * 16 + lanes) < nb
                m = jnp.logical_and(valid,
                                    lax.shift_right_logical(v, 22) ==
                                    lax.shift_right_logical(ct, 5))

                def while_cond(carry):
                    mm, _ = carry
                    return jnp.any(mm)

                def while_body(carry):
                    mm, h = carry
                    j = plsc.all_reduce_ffs(mm)[0]
                    comb = _extract(v, j)
                    cc = jnp.bitwise_and(
                        lax.shift_right_logical(comb, 15), 127)
                    slot = jnp.bitwise_and(comb, NSLOT - 1)
                    ring = lax.rem(h, 16)
                    ccv = jnp.full((16,), cc, jnp.int32)
                    for q4 in range(4):
                        g = plsc.load_gather(
                            piece.at[which],
                            [lanes + q4 * 16, ccv])
                        col[ring, pl.ds(q4 * 16, 16)] = g
                    pltpu.async_copy(
                        col.at[ring],
                        stage_hbm.at[pl.ds(slot * D, D)], csem)
                    @pl.when(h >= 16)
                    def _():
                        pltpu.make_async_copy(
                            stage_hbm.at[pl.ds(0, D)], col.at[0],
                            csem).wait()
                    return (jnp.logical_and(mm, lanes != j), h + 1)

                mm, h2 = lax.while_loop(while_cond, while_body, (m, hc2))
                return h2

            return lax.fori_loop(0, FBCAP // 16, vreg_hits, hc)

        return handle_hits(hitcount)

    total_hits = lax.fori_loop(0, CT_PER_W, ct_body, 0)

    def drain(i, c):
        @pl.when(i < jnp.minimum(total_hits, 16))
        def _():
            pltpu.make_async_copy(
                stage_hbm.at[pl.ds(0, D)], col.at[0], csem).wait()
        return c

    lax.fori_loop(0, 16, drain, 0)


@jax.jit
def _sweep_call(table_t, boxes, counts):
    f = pl.kernel(
        _sweep_body,
        mesh=plsc.VectorSubcoreMesh(core_axis_name="c", subcore_axis_name="s"),
        out_type=jax.ShapeDtypeStruct((NSLOT * D,), jnp.float32),
        scratch_types=[
            pltpu.VMEM((NW * BOXCAP,), jnp.int32),
            pltpu.VMEM((NW * NW,), jnp.int32),
            pltpu.VMEM((NBKT * BKTCAP,), jnp.int32),
            pltpu.VMEM((16,), jnp.int32),
            pltpu.VMEM((64 * FBCAP,), jnp.int32),
            pltpu.VMEM((64,), jnp.int32),
            pltpu.VMEM((4, D, 128), jnp.float32),
            pltpu.VMEM((16, D), jnp.float32),
            pltpu.SemaphoreType.DMA((4,)),
            pltpu.SemaphoreType.DMA,
        ],
        compiler_params=_PARAMS,
    )
    return f(table_t, boxes, counts)


def _dot_body(stage_hbm, wb_hbm, out_hbm, srows, trows, outv, wbv, sem):
    w = _wid()
    base = w * PPW
    pltpu.sync_copy(stage_hbm.at[pl.ds(base * D, PPW * D)], srows)
    pltpu.sync_copy(stage_hbm.at[pl.ds((B + base) * D, PPW * D)], trows)
    pltpu.sync_copy(wb_hbm, wbv)
    wv = wbv[...]
    wgt = wv[0]
    bb = wv[1]
    lanes = lax.iota(jnp.int32, 16)

    def group_body(g, carry):
        acc = jnp.zeros((16,), jnp.float32)
        for r in range(16):
            i = (g * 16 + r) * D
            a0 = srows[pl.ds(i, 16)] * trows[pl.ds(i, 16)]
            a1 = srows[pl.ds(i + 16, 16)] * trows[pl.ds(i + 16, 16)]
            a2 = srows[pl.ds(i + 32, 16)] * trows[pl.ds(i + 32, 16)]
            a3 = srows[pl.ds(i + 48, 16)] * trows[pl.ds(i + 48, 16)]
            s = jnp.sum((a0 + a1) + (a2 + a3))
            acc = jnp.where(lanes == r, s, acc)
        outv[pl.ds(g * 16, 16)] = acc * wgt + bb
        return carry

    lax.fori_loop(0, PPW // 16, group_body, 0)
    pltpu.sync_copy(outv, out_hbm.at[pl.ds(base, PPW)])


@jax.jit
def _dot_call(stage, wb):
    f = pl.kernel(
        _dot_body,
        mesh=plsc.VectorSubcoreMesh(core_axis_name="c", subcore_axis_name="s"),
        out_type=jax.ShapeDtypeStruct((B,), jnp.float32),
        scratch_types=[
            pltpu.VMEM((PPW * D,), jnp.float32),
            pltpu.VMEM((PPW * D,), jnp.float32),
            pltpu.VMEM((PPW,), jnp.float32),
            pltpu.VMEM((16,), jnp.float32),
            pltpu.SemaphoreType.DMA,
        ],
        compiler_params=_PARAMS,
    )
    return f(stage, wb)


def kernel(sources, targets, table, W, b):
    wb = jnp.zeros((16,), jnp.float32)
    wb = wb.at[0].set(W.reshape(())).at[1].set(b.reshape(()))
    boxes, counts = _bin_call(sources, targets)
    stage = _sweep_call(table.T, boxes, counts)
    out = _dot_call(stage, wb)
    return out.reshape(B, 1)


# speedup vs baseline: 1.2538x; 1.0001x over previous
import jax
import jax.numpy as jnp
from jax import lax
from jax.experimental import pallas as pl
from jax.experimental.pallas import tpu as pltpu
from jax.experimental.pallas import tpu_sc as plsc

B = 16384
D = 64
NW = 32
PPW = B // NW
CT_TOTAL = 7813
CT_PER_W = 245
BOXCAP = 128
NSLOT = 2 * B
_PARAMS = pltpu.CompilerParams(
    needs_layout_passes=False, use_tc_tiling_on_sc=True)


def _wid():
    return lax.axis_index("s") * 2 + lax.axis_index("c")


def _extract(vec, j):
    return vec[jnp.full((16,), j, jnp.int32)][0]


def _bin_body(src_hbm, tgt_hbm, boxes_hbm, counts_hbm, idxv, outbox, cntv,
              sem):
    w = _wid()
    pltpu.sync_copy(src_hbm.at[pl.ds(w * PPW, PPW)], idxv.at[pl.ds(0, PPW)])
    pltpu.sync_copy(tgt_hbm.at[pl.ds(w * PPW, PPW)],
                    idxv.at[pl.ds(PPW, PPW)])

    lanes = lax.iota(jnp.int32, 16)
    for k in range(NW):
        def scan_body(i, cnt, k=k):
            v = idxv[pl.ds(i * 16, 16)]
            ct = lax.shift_right_logical(v, 7)
            m = jnp.bitwise_and(ct, NW - 1) == k
            local = i * 16 + lanes
            slot = jnp.where(local < PPW, w * PPW + local,
                             B + w * PPW + (local - PPW))
            comb = (lax.shift_right_logical(ct, 5) << 22) | \
                   (jnp.bitwise_and(v, 127) << 15) | slot
            plsc.store_compressed(
                outbox.at[pl.ds(k * BOXCAP + cnt, 16)], comb, mask=m)
            return cnt + plsc.all_reduce_population_count(m)[0]

        cnt_k = lax.fori_loop(0, (2 * PPW) // 16, scan_body, 0)
        half = k // 16
        cntv[pl.ds(half * 16, 16)] = jnp.where(
            lanes == (k % 16), cnt_k, cntv[pl.ds(half * 16, 16)])

    pltpu.sync_copy(outbox, boxes_hbm.at[pl.ds(w * NW * BOXCAP, NW * BOXCAP)])
    pltpu.sync_copy(cntv, counts_hbm.at[pl.ds(w * NW, NW)])


@jax.jit
def _bin_call(sources, targets):
    f = pl.kernel(
        _bin_body,
        mesh=plsc.VectorSubcoreMesh(core_axis_name="c", subcore_axis_name="s"),
        out_type=(jax.ShapeDtypeStruct((NW * NW * BOXCAP,), jnp.int32),
                  jax.ShapeDtypeStruct((NW * NW,), jnp.int32)),
        scratch_types=[
            pltpu.VMEM((2 * PPW,), jnp.int32),
            pltpu.VMEM((NW * BOXCAP,), jnp.int32),
            pltpu.VMEM((NW,), jnp.int32),
            pltpu.SemaphoreType.DMA,
        ],
        compiler_params=_PARAMS,
    )
    return f(sources, targets)


WLCAP = 2048
NBKT = 16
BKTCAP = 128
FBCAP = 48


def _sweep_body(table_hbm, boxes_hbm, counts_hbm, stage_hbm,
                wl, cntv, bkt, bktcnt, fbkt, fbcnt, piece, col, sem, csem):
    k = _wid()
    lanes = lax.iota(jnp.int32, 16)

    pltpu.sync_copy(counts_hbm, cntv)
    for w in range(NW):
        pltpu.sync_copy(
            boxes_hbm.at[pl.ds((w * NW + k) * BOXCAP, BOXCAP)],
            wl.at[pl.ds(w * BOXCAP, BOXCAP)])

    def bucket_pass(b):
        def inner(w, cnt, b=b):
            mycnt = _extract(cntv[pl.ds((w * NW + k) // 16 * 16, 16)],
                             (w * NW + k) % 16)
            def vreg_body(q, cnt2):
                v = wl[pl.ds(w * BOXCAP + q * 16, 16)]
                valid = (q * 16 + lanes) < mycnt
                m = jnp.logical_and(
                    valid,
                    lax.shift_right_logical(v, 22 + 4) == b)
                plsc.store_compressed(
                    bkt.at[pl.ds(b * BKTCAP + cnt2, 16)], v, mask=m)
                return cnt2 + plsc.all_reduce_population_count(m)[0]
            return lax.fori_loop(0, BOXCAP // 16, vreg_body, cnt)
        return lax.fori_loop(0, NW, inner, 0)

    for b in range(NBKT):
        nb = bucket_pass(b)
        half = b // 16
        bktcnt[pl.ds(0, 16)] = jnp.where(lanes == b, nb, bktcnt[pl.ds(0, 16)])

    for b in range(NBKT):
        nbv = _extract(bktcnt[pl.ds(0, 16)], b)
        for sub in range(4):
            def sub_pass(q, cnt2, b=b, sub=sub, nbv=nbv):
                v = bkt[pl.ds(b * BKTCAP + q * 16, 16)]
                valid = (q * 16 + lanes) < nbv
                m = jnp.logical_and(
                    valid,
                    jnp.bitwise_and(lax.shift_right_logical(v, 24), 3) == sub)
                plsc.store_compressed(
                    fbkt.at[pl.ds((b * 4 + sub) * FBCAP + cnt2, 16)],
                    v, mask=m)
                return cnt2 + plsc.all_reduce_population_count(m)[0]
            nf = lax.fori_loop(0, BKTCAP // 16, sub_pass, 0)
            fb = b * 4 + sub
            half2 = fb // 16
            fbcnt[pl.ds(half2 * 16, 16)] = jnp.where(
                lanes == (fb % 16), nf, fbcnt[pl.ds(half2 * 16, 16)])

    def fire(ct_i, which):
        ct = ct_i * NW + k
        @pl.when(ct < CT_TOTAL - 1)
        def _():
            for bb in range(8):
                pltpu.async_copy(
                    table_hbm.at[pl.ds(bb * 8, 8), pl.ds(ct * 128, 128)],
                    piece.at[which, pl.ds(bb * 8, 8), :], sem.at[which])
        @pl.when(ct == CT_TOTAL - 1)
        def _():
            for bb in range(8):
                pltpu.async_copy(
                    table_hbm.at[pl.ds(bb * 8, 8), pl.ds(ct * 128, 64)],
                    piece.at[which, pl.ds(bb * 8, 8), pl.ds(0, 64)],
                    sem.at[which])

    for pf in range(7):
        fire(pf, pf)

    def ct_body(ct_i, hitcount):
        which = lax.rem(ct_i, 8)
        ct = ct_i * NW + k

        @pl.when(ct_i + 7 < CT_PER_W)
        def _():
            fire(ct_i + 7, lax.rem(ct_i + 7, 8))

        @pl.when(ct < CT_TOTAL - 1)
        def _():
            for bb in range(8):
                pltpu.make_async_copy(
                    table_hbm.at[pl.ds(0, 8), pl.ds(0, 128)],
                    piece.at[0].at[pl.ds(bb * 8, 8), :],
                    sem.at[which]).wait()

        @pl.when(ct == CT_TOTAL - 1)
        def _():
            for bb in range(8):
                pltpu.make_async_copy(
                    table_hbm.at[pl.ds(0, 8), pl.ds(0, 64)],
                    piece.at[0].at[pl.ds(bb * 8, 8), pl.ds(0, 64)],
                    sem.at[which]).wait()

        def handle_hits(hc):
            b = lax.shift_right_logical(ct_i, 2)
            nb = _extract(
                fbcnt[pl.ds(lax.shift_right_logical(b, 4) * 16, 16)],
                jnp.bitwise_and(b, 15))

            def vreg_hits(q, hc2):
                v = fbkt[pl.ds(b * FBCAP + q * 16, 16)]
                valid = (q * 16 + lanes) < nb
                m = jnp.logical_and(valid,
                                    lax.shift_right_logical(v, 22) ==
                                    lax.shift_right_logical(ct, 5))

                def while_cond(carry):
                    mm, _ = carry
                    return jnp.any(mm)

                def while_body(carry):
                    mm, h = carry
                    j = plsc.all_reduce_ffs(mm)[0]
                    comb = _extract(v, j)
                    cc = jnp.bitwise_and(
                        lax.shift_right_logical(comb, 15), 127)
                    slot = jnp.bitwise_and(comb, NSLOT - 1)
                    ring = lax.rem(h, 16)
                    ccv = jnp.full((16,), cc, jnp.int32)
                    for q4 in range(4):
                        g = plsc.load_gather(
                            piece.at[which],
                            [lanes + q4 * 16, ccv])
                        col[ring, pl.ds(q4 * 16, 16)] = g
                    pltpu.async_copy(
                        col.at[ring],
                        stage_hbm.at[pl.ds(slot * D, D)], csem)
                    @pl.when(h >= 16)
                    def _():
                        pltpu.make_async_copy(
                            stage_hbm.at[pl.ds(0, D)], col.at[0],
                            csem).wait()
                    return (jnp.logical_and(mm, lanes != j), h + 1)

                mm, h2 = lax.while_loop(while_cond, while_body, (m, hc2))
                return h2

            return lax.fori_loop(0, FBCAP // 16, vreg_hits, hc)

        return handle_hits(hitcount)

    total_hits = lax.fori_loop(0, CT_PER_W, ct_body, 0)

    def drain(i, c):
        @pl.when(i < jnp.minimum(total_hits, 16))
        def _():
            pltpu.make_async_copy(
                stage_hbm.at[pl.ds(0, D)], col.at[0], csem).wait()
        return c

    lax.fori_loop(0, 16, drain, 0)


@jax.jit
def _sweep_call(table_t, boxes, counts):
    f = pl.kernel(
        _sweep_body,
        mesh=plsc.VectorSubcoreMesh(core_axis_name="c", subcore_axis_name="s"),
        out_type=jax.ShapeDtypeStruct((NSLOT * D,), jnp.float32),
        scratch_types=[
            pltpu.VMEM((NW * BOXCAP,), jnp.int32),
            pltpu.VMEM((NW * NW,), jnp.int32),
            pltpu.VMEM((NBKT * BKTCAP,), jnp.int32),
            pltpu.VMEM((16,), jnp.int32),
            pltpu.VMEM((64 * FBCAP,), jnp.int32),
            pltpu.VMEM((64,), jnp.int32),
            pltpu.VMEM((8, D, 128), jnp.float32),
            pltpu.VMEM((16, D), jnp.float32),
            pltpu.SemaphoreType.DMA((8,)),
            pltpu.SemaphoreType.DMA,
        ],
        compiler_params=_PARAMS,
    )
    return f(table_t, boxes, counts)


def _dot_body(stage_hbm, wb_hbm, out_hbm, srows, trows, outv, wbv, sem):
    w = _wid()
    base = w * PPW
    pltpu.sync_copy(stage_hbm.at[pl.ds(base * D, PPW * D)], srows)
    pltpu.sync_copy(stage_hbm.at[pl.ds((B + base) * D, PPW * D)], trows)
    pltpu.sync_copy(wb_hbm, wbv)
    wv = wbv[...]
    wgt = wv[0]
    bb = wv[1]
    lanes = lax.iota(jnp.int32, 16)

    def group_body(g, carry):
        acc = jnp.zeros((16,), jnp.float32)
        for r in range(16):
            i = (g * 16 + r) * D
            a0 = srows[pl.ds(i, 16)] * trows[pl.ds(i, 16)]
            a1 = srows[pl.ds(i + 16, 16)] * trows[pl.ds(i + 16, 16)]
            a2 = srows[pl.ds(i + 32, 16)] * trows[pl.ds(i + 32, 16)]
            a3 = srows[pl.ds(i + 48, 16)] * trows[pl.ds(i + 48, 16)]
            s = jnp.sum((a0 + a1) + (a2 + a3))
            acc = jnp.where(lanes == r, s, acc)
        outv[pl.ds(g * 16, 16)] = acc * wgt + bb
        return carry

    lax.fori_loop(0, PPW // 16, group_body, 0)
    pltpu.sync_copy(outv, out_hbm.at[pl.ds(base, PPW)])


@jax.jit
def _dot_call(stage, wb):
    f = pl.kernel(
        _dot_body,
        mesh=plsc.VectorSubcoreMesh(core_axis_name="c", subcore_axis_name="s"),
        out_type=jax.ShapeDtypeStruct((B,), jnp.float32),
        scratch_types=[
            pltpu.VMEM((PPW * D,), jnp.float32),
            pltpu.VMEM((PPW * D,), jnp.float32),
            pltpu.VMEM((PPW,), jnp.float32),
            pltpu.VMEM((16,), jnp.float32),
            pltpu.SemaphoreType.DMA,
        ],
        compiler_params=_PARAMS,
    )
    return f(stage, wb)


def kernel(sources, targets, table, W, b):
    wb = jnp.zeros((16,), jnp.float32)
    wb = wb.at[0].set(W.reshape(())).at[1].set(b.reshape(()))
    boxes, counts = _bin_call(sources, targets)
    stage = _sweep_call(table.T, boxes, counts)
    out = _dot_call(stage, wb)
    return out.reshape(B, 1)
